# baseline (speedup 1.0000x reference)
import jax
import jax.numpy as jnp
from jax.experimental import pallas as pl

_LANE_BLOCK = 32768


def _scale_kernel(w_ref, t_ref, o_ref):
    o_ref[0] = t_ref[0]
    pw = jnp.maximum(w_ref[...], 0.0) + 1e-9
    o_ref[1] = t_ref[1] * pw


def kernel(t, weights):
    v, _, width = t.shape
    tt = jnp.transpose(t, (1, 2, 0))
    wt = jnp.transpose(weights, (1, 0))
    c = min(_LANE_BLOCK, v)
    g = -(-v // c)
    out = pl.pallas_call(
        _scale_kernel,
        grid=(g,),
        in_specs=[
            pl.BlockSpec((width, c), lambda i: (0, i)),
            pl.BlockSpec((2, width, c), lambda i: (0, 0, i)),
        ],
        out_specs=pl.BlockSpec((2, width, c), lambda i: (0, 0, i)),
        out_shape=jax.ShapeDtypeStruct((2, width, v), jnp.float32),
    )(wt, tt)
    return jnp.transpose(out, (2, 0, 1))
